# baseline (speedup 1.0000x reference)
import jax
import jax.numpy as jnp
from jax.experimental import pallas as pl

_N_NEUR = 16384
_NUM_MD = 1024
_ROW_BLOCK = 64
_N_BLOCKS = _NUM_MD // _ROW_BLOCK
_ALPHA = 0.001 / (0.02 * 4)


def _md_kernel(x_ref, w_ref, md_ref, out_ref):
    i = pl.program_id(0)
    x = x_ref[...]
    w = w_ref[...]
    mv = jnp.sum(w * x, axis=1).reshape(1, _ROW_BLOCK)
    md = md_ref[pl.ds(i, 1), :]
    out_ref[pl.ds(i, 1), :] = md * (1.0 - _ALPHA) + _ALPHA * mv

    @pl.when(i == _N_BLOCKS - 1)
    def _wta():
        v = out_ref[...]
        m1 = jnp.max(v)
        is_max = v == m1
        cnt = jnp.sum(is_max.astype(jnp.float32))
        m2 = jnp.max(jnp.where(is_max, jnp.finfo(jnp.float32).min, v))
        thr = jnp.where(cnt >= 2.0, m1, (m1 + m2) * 0.5)
        out_ref[...] = jnp.where(v >= thr, 1.0, 0.0)


def kernel(input, wPFC2MD, MDinp):
    x2 = input.reshape(1, _N_NEUR)
    md2 = MDinp.reshape(_N_BLOCKS, _ROW_BLOCK)
    out = pl.pallas_call(
        _md_kernel,
        grid=(_N_BLOCKS,),
        in_specs=[
            pl.BlockSpec((1, _N_NEUR), lambda i: (0, 0)),
            pl.BlockSpec((_ROW_BLOCK, _N_NEUR), lambda i: (i, 0)),
            pl.BlockSpec((_N_BLOCKS, _ROW_BLOCK), lambda i: (0, 0)),
        ],
        out_specs=pl.BlockSpec((_N_BLOCKS, _ROW_BLOCK), lambda i: (0, 0)),
        out_shape=jax.ShapeDtypeStruct((_N_BLOCKS, _ROW_BLOCK), jnp.float32),
    )(x2, wPFC2MD, md2)
    return out.reshape(_NUM_MD)


# speedup vs baseline: 1.2396x; 1.2396x over previous
import jax
import jax.numpy as jnp
from jax.experimental import pallas as pl

_N_NEUR = 16384
_NUM_MD = 1024
_ROW_BLOCK = 128
_N_BLOCKS = _NUM_MD // _ROW_BLOCK
_ALPHA = 0.001 / (0.02 * 4)


def _md_kernel(x_ref, w_ref, md_ref, out_ref):
    i = pl.program_id(0)
    x = x_ref[...]
    w = w_ref[...]
    mv = jnp.sum(w * x, axis=1).reshape(1, _ROW_BLOCK)
    md = md_ref[pl.ds(i, 1), :]
    out_ref[pl.ds(i, 1), :] = md * (1.0 - _ALPHA) + _ALPHA * mv

    @pl.when(i == _N_BLOCKS - 1)
    def _wta():
        v = out_ref[...]
        m1 = jnp.max(v)
        is_max = v == m1
        cnt = jnp.sum(is_max.astype(jnp.float32))
        m2 = jnp.max(jnp.where(is_max, jnp.finfo(jnp.float32).min, v))
        thr = jnp.where(cnt >= 2.0, m1, (m1 + m2) * 0.5)
        out_ref[...] = jnp.where(v >= thr, 1.0, 0.0)


def kernel(input, wPFC2MD, MDinp):
    x2 = input.reshape(1, _N_NEUR)
    md2 = MDinp.reshape(_N_BLOCKS, _ROW_BLOCK)
    out = pl.pallas_call(
        _md_kernel,
        grid=(_N_BLOCKS,),
        in_specs=[
            pl.BlockSpec((1, _N_NEUR), lambda i: (0, 0)),
            pl.BlockSpec((_ROW_BLOCK, _N_NEUR), lambda i: (i, 0)),
            pl.BlockSpec((_N_BLOCKS, _ROW_BLOCK), lambda i: (0, 0)),
        ],
        out_specs=pl.BlockSpec((_N_BLOCKS, _ROW_BLOCK), lambda i: (0, 0)),
        out_shape=jax.ShapeDtypeStruct((_N_BLOCKS, _ROW_BLOCK), jnp.float32),
    )(x2, wPFC2MD, md2)
    return out.reshape(_NUM_MD)
